# baseline (speedup 1.0000x reference)
import functools

import jax
import jax.numpy as jnp
from jax.experimental import pallas as pl
from jax.experimental.pallas import tpu as pltpu

N, D_IN, D_OUT, E = 4096, 2048, 2048, 8
BLOCK_O = 256
CHUNK_N = 512


def _moe_kernel(x_ref, gw_ref, gb_ref, ew_ref, eb_ref, out_ref, w8_ref):
    o = pl.program_id(0)
    e = pl.program_id(1)

    @pl.when(jnp.logical_and(o == 0, e == 0))
    def _():
        gs = jax.lax.dot_general(
            x_ref[...], gw_ref[...], (((1,), (1,)), ((), ())),
            preferred_element_type=jnp.float32,
        ) + gb_ref[...]
        lane = jax.lax.broadcasted_iota(jnp.int32, gs.shape, 1)
        m1 = jnp.max(gs, axis=1, keepdims=True)
        i1 = jnp.min(jnp.where(gs == m1, lane, E), axis=1, keepdims=True)
        masked = jnp.where(lane == i1, -jnp.inf, gs)
        m2 = jnp.max(masked, axis=1, keepdims=True)
        i2 = jnp.min(jnp.where(masked == m2, lane, E), axis=1, keepdims=True)
        w1 = 1.0 / (1.0 + jnp.exp(m2 - m1))
        w8_ref[...] = jnp.where(
            lane == i1, w1, jnp.where(lane == i2, 1.0 - w1, 0.0)
        )

    def body(i, carry):
        sl = pl.ds(i * CHUNK_N, CHUNK_N)
        xs = x_ref[sl, :]
        y = jax.lax.dot_general(
            xs, ew_ref[0], (((1,), (1,)), ((), ())),
            preferred_element_type=jnp.float32,
        ) + eb_ref[0]
        lane = jax.lax.broadcasted_iota(jnp.int32, (CHUNK_N, E), 1)
        we = jnp.sum(w8_ref[sl, :] * (lane == e), axis=1, keepdims=True)
        val = we * y

        @pl.when(e == 0)
        def _():
            out_ref[sl, :] = val

        @pl.when(e != 0)
        def _():
            out_ref[sl, :] += val

        return carry

    jax.lax.fori_loop(0, N // CHUNK_N, body, 0)


@jax.jit
def kernel(x, gate_W, gate_b, expert_W, expert_b):
    grid = (D_OUT // BLOCK_O, E)
    return pl.pallas_call(
        _moe_kernel,
        grid=grid,
        in_specs=[
            pl.BlockSpec((N, D_IN), lambda o, e: (0, 0)),
            pl.BlockSpec((E, D_IN), lambda o, e: (0, 0)),
            pl.BlockSpec((1, E), lambda o, e: (0, 0)),
            pl.BlockSpec((1, BLOCK_O, D_IN), lambda o, e: (e, o, 0)),
            pl.BlockSpec((1, 1, BLOCK_O), lambda o, e: (e, 0, o)),
        ],
        out_specs=pl.BlockSpec((N, BLOCK_O), lambda o, e: (0, o)),
        out_shape=jax.ShapeDtypeStruct((N, D_OUT), jnp.float32),
        scratch_shapes=[pltpu.VMEM((N, E), jnp.float32)],
        compiler_params=pltpu.CompilerParams(
            dimension_semantics=("arbitrary", "arbitrary"),
        ),
    )(x, gate_W, gate_b.reshape(1, E), expert_W, expert_b.reshape(E, 1, D_OUT))
